# baseline (speedup 1.0000x reference)
import functools

import jax
import jax.numpy as jnp
from jax import lax
from jax.experimental import pallas as pl
from jax.experimental.pallas import tpu as pltpu
from jax.experimental.pallas import tpu_sc as plsc

N = 10000
E = 320000
D = 128
DE = 16
H = 32

NC = 2
NS = 16
NW = NC * NS
EPW = E // NW
CHUNK = 125
NCHUNK = EPW // CHUNK
NPAD = 10240
SLICE = NPAD // NS

BN = 1000


def _proj_body(x_ref, wa_ref, wb_ref, xs_ref, xd_ref):
    xb = x_ref[...]
    xs_ref[...] = jnp.dot(xb, wa_ref[...], preferred_element_type=jnp.float32)
    xd_ref[...] = jnp.dot(xb, wb_ref[...], preferred_element_type=jnp.float32)


def _project_nodes(x, w1a, w1b):
    return pl.pallas_call(
        _proj_body,
        grid=(N // BN,),
        in_specs=[
            pl.BlockSpec((BN, D), lambda i: (i, 0)),
            pl.BlockSpec((D, H), lambda i: (0, 0)),
            pl.BlockSpec((D, H), lambda i: (0, 0)),
        ],
        out_specs=[
            pl.BlockSpec((BN, H), lambda i: (i, 0)),
            pl.BlockSpec((BN, H), lambda i: (i, 0)),
        ],
        out_shape=[
            jax.ShapeDtypeStruct((N, H), jnp.float32),
            jax.ShapeDtypeStruct((N, H), jnp.float32),
        ],
    )(x, w1a, w1b)


def _gather_body(xs_hbm, xd_hbm, src3, dst3, gs_hbm, gd_hbm,
                 idxs, idxd, rows_a, rows_b, sem_a, sem_b):
    c = lax.axis_index("c")
    s = lax.axis_index("s")
    wid = s * NC + c
    pltpu.sync_copy(src3.at[wid], idxs)
    pltpu.sync_copy(dst3.at[wid], idxd)

    def step(j, carry):
        cp_a = pltpu.async_copy(xs_hbm.at[idxs.at[j]], rows_a, sem_a)
        cp_b = pltpu.async_copy(xd_hbm.at[idxd.at[j]], rows_b, sem_b)
        cp_a.wait()
        cp_b.wait()
        pltpu.sync_copy(rows_a, gs_hbm.at[wid, j])
        pltpu.sync_copy(rows_b, gd_hbm.at[wid, j])
        return carry

    lax.fori_loop(0, NCHUNK, step, 0)


def _gather_edges(xs, xd, src3, dst3):
    mesh = plsc.VectorSubcoreMesh(core_axis_name="c", subcore_axis_name="s", num_cores=NC, num_subcores=NS)
    k = functools.partial(
        pl.kernel,
        mesh=mesh,
        compiler_params=pltpu.CompilerParams(use_tc_tiling_on_sc=False),
        out_type=[
            jax.ShapeDtypeStruct((NW, NCHUNK, CHUNK, H), jnp.float32),
            jax.ShapeDtypeStruct((NW, NCHUNK, CHUNK, H), jnp.float32),
        ],
        scratch_types=[
            pltpu.VMEM((NCHUNK, CHUNK), jnp.int32),
            pltpu.VMEM((NCHUNK, CHUNK), jnp.int32),
            pltpu.VMEM((CHUNK, H), jnp.float32),
            pltpu.VMEM((CHUNK, H), jnp.float32),
            pltpu.SemaphoreType.DMA,
            pltpu.SemaphoreType.DMA,
        ],
    )(_gather_body)
    return k(xs, xd, src3, dst3)


M = E // 4
BP = 3200


def _mlp_body(gs_ref, gd_ref, ea_ref, w1c4_ref, b1t_ref, w240_ref,
              w241_ref, w2d4_ref, bs_ref, p0_ref, p1_ref, ind_ref):
    g = gs_ref[...] + gd_ref[...]
    pre = g + jnp.dot(ea_ref[...], w1c4_ref[...],
                      preferred_element_type=jnp.float32) + b1t_ref[...]
    h = jnp.maximum(pre, 0.0)
    tdims = (((0,), (1,)), ((), ()))
    p0_ref[...] = lax.dot_general(w240_ref[...], h, tdims,
                                  preferred_element_type=jnp.float32) + bs_ref[0, 0]
    p1_ref[...] = lax.dot_general(w241_ref[...], h, tdims,
                                  preferred_element_type=jnp.float32) + bs_ref[0, 1]
    d = lax.dot_general(w2d4_ref[...], h, tdims,
                        preferred_element_type=jnp.float32) + bs_ref[0, 2]
    score = jax.nn.sigmoid(d)
    ind_ref[...] = (score > 0.9).astype(jnp.float32)


def _edge_mlp(gs_p, gd_p, ea4, w1c4, b1t, w240, w241, w2d4, bs):
    return pl.pallas_call(
        _mlp_body,
        grid=(M // BP,),
        in_specs=[
            pl.BlockSpec((BP, 128), lambda i: (i, 0)),
            pl.BlockSpec((BP, 128), lambda i: (i, 0)),
            pl.BlockSpec((BP, 64), lambda i: (i, 0)),
            pl.BlockSpec((64, 128), lambda i: (0, 0)),
            pl.BlockSpec((1, 128), lambda i: (0, 0)),
            pl.BlockSpec((128, 4), lambda i: (0, 0)),
            pl.BlockSpec((128, 4), lambda i: (0, 0)),
            pl.BlockSpec((128, 4), lambda i: (0, 0)),
            pl.BlockSpec((1, 4), lambda i: (0, 0)),
        ],
        out_specs=[
            pl.BlockSpec((4, BP), lambda i: (0, i)),
            pl.BlockSpec((4, BP), lambda i: (0, i)),
            pl.BlockSpec((4, BP), lambda i: (0, i)),
        ],
        out_shape=[
            jax.ShapeDtypeStruct((4, M), jnp.float32),
            jax.ShapeDtypeStruct((4, M), jnp.float32),
            jax.ShapeDtypeStruct((4, M), jnp.float32),
        ],
    )(gs_p, gd_p, ea4, w1c4, b1t, w240, w241, w2d4, bs)


def _scatter_body(dst2, ind2, out_hbm, dstb, indb, hist, shared, tmprow, acc):
    c = lax.axis_index("c")
    s = lax.axis_index("s")
    wid = s * NC + c
    pltpu.sync_copy(dst2.at[wid], dstb)
    pltpu.sync_copy(ind2.at[wid], indb)

    zeros16 = jnp.zeros((16,), jnp.float32)
    ones16 = jnp.ones((16,), jnp.float32)

    def zero_step(i, carry):
        hist[pl.ds(i * 16, 16)] = zeros16
        return carry

    lax.fori_loop(0, NPAD // 16, zero_step, 0)

    def scat_step(i, carry):
        idx = dstb[pl.ds(i * 16, 16)]
        v = indb[pl.ds(i * 16, 16)]
        plsc.store_scatter(hist, [idx], ones16, mask=v > 0.5)
        return carry

    lax.fori_loop(0, EPW // 16, scat_step, 0)

    pltpu.sync_copy(hist, shared.at[s])
    plsc.subcore_barrier()

    off = s * SLICE

    def zero_acc(t, carry):
        acc[pl.ds(t * 16, 16)] = zeros16
        return carry

    lax.fori_loop(0, SLICE // 16, zero_acc, 0)

    def red_row(w, carry):
        pltpu.sync_copy(shared.at[w, pl.ds(off, SLICE)], tmprow)

        def red_col(t, carry2):
            sl = pl.ds(t * 16, 16)
            acc[sl] = jnp.maximum(acc[sl], tmprow[sl])
            return carry2

        lax.fori_loop(0, SLICE // 16, red_col, 0)
        return carry

    lax.fori_loop(0, NS, red_row, 0)
    pltpu.sync_copy(acc, out_hbm.at[c, pl.ds(off, SLICE)])


def _segment_any(dst2, ind2):
    mesh = plsc.VectorSubcoreMesh(core_axis_name="c", subcore_axis_name="s", num_cores=NC, num_subcores=NS)
    k = functools.partial(
        pl.kernel,
        mesh=mesh,
        compiler_params=pltpu.CompilerParams(
            use_tc_tiling_on_sc=False, needs_layout_passes=False),
        out_type=jax.ShapeDtypeStruct((NC, NPAD), jnp.float32),
        scratch_types=[
            pltpu.VMEM((EPW,), jnp.int32),
            pltpu.VMEM((EPW,), jnp.float32),
            pltpu.VMEM((NPAD,), jnp.float32),
            pltpu.VMEM_SHARED((NS, NPAD), jnp.float32),
            pltpu.VMEM((SLICE,), jnp.float32),
            pltpu.VMEM((SLICE,), jnp.float32),
        ],
    )(_scatter_body)
    return k(dst2, ind2)


def _combine_body(p_ref, out_ref):
    p = p_ref[...]
    m = jnp.maximum(p[0:1, :], p[1:2, :])
    out_ref[...] = (m > 0.5).astype(jnp.float32)


def _combine(partials):
    return pl.pallas_call(
        _combine_body,
        grid=(1,),
        in_specs=[pl.BlockSpec((NC, NPAD), lambda i: (0, 0))],
        out_specs=pl.BlockSpec((1, NPAD), lambda i: (0, 0)),
        out_shape=jax.ShapeDtypeStruct((1, NPAD), jnp.float32),
    )(partials)


def kernel(x, edge_index, edge_attr, W1, b1, W2, b2):
    x = x.astype(jnp.float32)
    edge_attr = edge_attr.astype(jnp.float32)
    src = edge_index[0].astype(jnp.int32)
    dst = edge_index[1].astype(jnp.int32)

    w1a = W1[:D]
    w1b = W1[D:2 * D]
    w1c = W1[2 * D:]
    w2d = (W2[:, 1] - W2[:, 0]).reshape(H, 1)
    b2d = b2[1] - b2[0]

    eye4 = jnp.eye(4, dtype=jnp.float32)
    w1c4 = jnp.kron(eye4, w1c)
    w240 = jnp.kron(eye4, W2[:, 0:1])
    w241 = jnp.kron(eye4, W2[:, 1:2])
    w2d4 = jnp.kron(eye4, w2d)
    b1t = jnp.tile(b1, 4).reshape(1, 128)
    bs = jnp.stack([b2[0], b2[1], b2d, b2d]).reshape(1, 4)

    xs, xd = _project_nodes(x, w1a, w1b)

    src3 = src.reshape(NW, NCHUNK, CHUNK)
    dst3 = dst.reshape(NW, NCHUNK, CHUNK)
    gs4, gd4 = _gather_edges(xs, xd, src3, dst3)
    gs_p = gs4.reshape(M, 128)
    gd_p = gd4.reshape(M, 128)
    ea4 = edge_attr.reshape(M, 4 * DE)

    p0t, p1t, indt = _edge_mlp(gs_p, gd_p, ea4,
                               w1c4, b1t, w240, w241, w2d4, bs)
    edge_pred = jnp.stack([p0t.T.reshape(E), p1t.T.reshape(E)], axis=1)

    dst_perm = dst.reshape(M, 4).T.reshape(NW, EPW)
    ind2 = indt.reshape(NW, EPW)
    partials = _segment_any(dst_perm, ind2)

    matched01 = _combine(partials)
    matched = matched01[0, :N] != 0.0
    return edge_pred, matched


# speedup vs baseline: 1.7364x; 1.7364x over previous
import functools

import jax
import jax.numpy as jnp
from jax import lax
from jax.experimental import pallas as pl
from jax.experimental.pallas import tpu as pltpu
from jax.experimental.pallas import tpu_sc as plsc

N = 10000
E = 320000
D = 128
DE = 16
H = 32

NC = 2
NS = 16
NW = NC * NS
EPW = E // NW
CHUNK = 125
NCHUNK = EPW // CHUNK
NPAD = 10240
SLICE = NPAD // NS
M = E // 4

BN = 1000


def _proj_body(x_ref, wa_ref, wb_ref, xs_ref, xd_ref):
    xb = x_ref[...]
    xs_ref[...] = jnp.dot(xb, wa_ref[...], preferred_element_type=jnp.float32)
    xd_ref[...] = jnp.dot(xb, wb_ref[...], preferred_element_type=jnp.float32)


def _project_nodes(x, w1a, w1b):
    return pl.pallas_call(
        _proj_body,
        grid=(N // BN,),
        in_specs=[
            pl.BlockSpec((BN, D), lambda i: (i, 0)),
            pl.BlockSpec((D, H), lambda i: (0, 0)),
            pl.BlockSpec((D, H), lambda i: (0, 0)),
        ],
        out_specs=[
            pl.BlockSpec((BN, H), lambda i: (i, 0)),
            pl.BlockSpec((BN, H), lambda i: (i, 0)),
        ],
        out_shape=[
            jax.ShapeDtypeStruct((N, H), jnp.float32),
            jax.ShapeDtypeStruct((N, H), jnp.float32),
        ],
    )(x, w1a, w1b)


MPW = M // NW
PCHUNK = 125
NPCH = MPW // PCHUNK


def _gather_body(xs_hbm, xd_hbm, src4, dst4, gs_hbm, gd_hbm,
                 idxs, idxd, tmp_a, tmp_b, sem_a, sem_b):
    c = lax.axis_index("c")
    s = lax.axis_index("s")
    wid = s * NC + c
    pltpu.sync_copy(src4.at[:, wid], idxs)
    pltpu.sync_copy(dst4.at[:, wid], idxd)

    def step(j, carry):
        for a in range(4):
            cp_a = pltpu.async_copy(xs_hbm.at[idxs.at[a, j]], tmp_a, sem_a)
            cp_b = pltpu.async_copy(xd_hbm.at[idxd.at[a, j]], tmp_b, sem_b)
            cp_a.wait()
            cp_b.wait()
            pltpu.sync_copy(tmp_a, gs_hbm.at[wid, j, :, a])
            pltpu.sync_copy(tmp_b, gd_hbm.at[wid, j, :, a])
        return carry

    lax.fori_loop(0, NPCH, step, 0)


def _gather_edges(xs, xd, src4, dst4):
    mesh = plsc.VectorSubcoreMesh(core_axis_name="c", subcore_axis_name="s", num_cores=NC, num_subcores=NS)
    k = functools.partial(
        pl.kernel,
        mesh=mesh,
        compiler_params=pltpu.CompilerParams(use_tc_tiling_on_sc=False),
        out_type=[
            jax.ShapeDtypeStruct((NW, NPCH, PCHUNK, 4, H), jnp.float32),
            jax.ShapeDtypeStruct((NW, NPCH, PCHUNK, 4, H), jnp.float32),
        ],
        scratch_types=[
            pltpu.VMEM((4, NPCH, PCHUNK), jnp.int32),
            pltpu.VMEM((4, NPCH, PCHUNK), jnp.int32),
            pltpu.VMEM((PCHUNK, H), jnp.float32),
            pltpu.VMEM((PCHUNK, H), jnp.float32),
            pltpu.SemaphoreType.DMA,
            pltpu.SemaphoreType.DMA,
        ],
    )(_gather_body)
    return k(xs, xd, src4, dst4)


BP = 3200


def _mlp_body(gs_ref, gd_ref, ea_ref, w1c4_ref, b1t_ref, w240_ref,
              w241_ref, w2d4_ref, bs_ref, p0_ref, p1_ref, ind_ref):
    g = gs_ref[...] + gd_ref[...]
    pre = g + jnp.dot(ea_ref[...], w1c4_ref[...],
                      preferred_element_type=jnp.float32) + b1t_ref[...]
    h = jnp.maximum(pre, 0.0)
    tdims = (((0,), (1,)), ((), ()))
    p0_ref[...] = lax.dot_general(w240_ref[...], h, tdims,
                                  preferred_element_type=jnp.float32) + bs_ref[0, 0]
    p1_ref[...] = lax.dot_general(w241_ref[...], h, tdims,
                                  preferred_element_type=jnp.float32) + bs_ref[0, 1]
    d = lax.dot_general(w2d4_ref[...], h, tdims,
                        preferred_element_type=jnp.float32) + bs_ref[0, 2]
    score = jax.nn.sigmoid(d)
    ind_ref[...] = (score > 0.9).astype(jnp.float32)


def _edge_mlp(gs_p, gd_p, ea4, w1c4, b1t, w240, w241, w2d4, bs):
    return pl.pallas_call(
        _mlp_body,
        grid=(M // BP,),
        in_specs=[
            pl.BlockSpec((BP, 128), lambda i: (i, 0)),
            pl.BlockSpec((BP, 128), lambda i: (i, 0)),
            pl.BlockSpec((BP, 64), lambda i: (i, 0)),
            pl.BlockSpec((64, 128), lambda i: (0, 0)),
            pl.BlockSpec((1, 128), lambda i: (0, 0)),
            pl.BlockSpec((128, 4), lambda i: (0, 0)),
            pl.BlockSpec((128, 4), lambda i: (0, 0)),
            pl.BlockSpec((128, 4), lambda i: (0, 0)),
            pl.BlockSpec((1, 4), lambda i: (0, 0)),
        ],
        out_specs=[
            pl.BlockSpec((4, BP), lambda i: (0, i)),
            pl.BlockSpec((4, BP), lambda i: (0, i)),
            pl.BlockSpec((4, BP), lambda i: (0, i)),
        ],
        out_shape=[
            jax.ShapeDtypeStruct((4, M), jnp.float32),
            jax.ShapeDtypeStruct((4, M), jnp.float32),
            jax.ShapeDtypeStruct((4, M), jnp.float32),
        ],
    )(gs_p, gd_p, ea4, w1c4, b1t, w240, w241, w2d4, bs)


def _scatter_body(dst2, ind2, out_hbm, dstb, indb, hist, shared, tmprow, acc):
    c = lax.axis_index("c")
    s = lax.axis_index("s")
    wid = s * NC + c
    pltpu.sync_copy(dst2.at[wid], dstb)
    pltpu.sync_copy(ind2.at[wid], indb)

    zeros16 = jnp.zeros((16,), jnp.float32)
    ones16 = jnp.ones((16,), jnp.float32)

    def zero_step(i, carry):
        hist[pl.ds(i * 16, 16)] = zeros16
        return carry

    lax.fori_loop(0, NPAD // 16, zero_step, 0)

    def scat_step(i, carry):
        idx = dstb[pl.ds(i * 16, 16)]
        v = indb[pl.ds(i * 16, 16)]
        plsc.store_scatter(hist, [idx], ones16, mask=v > 0.5)
        return carry

    lax.fori_loop(0, EPW // 16, scat_step, 0)

    pltpu.sync_copy(hist, shared.at[s])
    plsc.subcore_barrier()

    off = s * SLICE

    def zero_acc(t, carry):
        acc[pl.ds(t * 16, 16)] = zeros16
        return carry

    lax.fori_loop(0, SLICE // 16, zero_acc, 0)

    def red_row(w, carry):
        pltpu.sync_copy(shared.at[w, pl.ds(off, SLICE)], tmprow)

        def red_col(t, carry2):
            sl = pl.ds(t * 16, 16)
            acc[sl] = jnp.maximum(acc[sl], tmprow[sl])
            return carry2

        lax.fori_loop(0, SLICE // 16, red_col, 0)
        return carry

    lax.fori_loop(0, NS, red_row, 0)
    pltpu.sync_copy(acc, out_hbm.at[c, pl.ds(off, SLICE)])


def _segment_any(dst2, ind2):
    mesh = plsc.VectorSubcoreMesh(core_axis_name="c", subcore_axis_name="s", num_cores=NC, num_subcores=NS)
    k = functools.partial(
        pl.kernel,
        mesh=mesh,
        compiler_params=pltpu.CompilerParams(
            use_tc_tiling_on_sc=False, needs_layout_passes=False),
        out_type=jax.ShapeDtypeStruct((NC, NPAD), jnp.float32),
        scratch_types=[
            pltpu.VMEM((EPW,), jnp.int32),
            pltpu.VMEM((EPW,), jnp.float32),
            pltpu.VMEM((NPAD,), jnp.float32),
            pltpu.VMEM_SHARED((NS, NPAD), jnp.float32),
            pltpu.VMEM((SLICE,), jnp.float32),
            pltpu.VMEM((SLICE,), jnp.float32),
        ],
    )(_scatter_body)
    return k(dst2, ind2)


def _combine_body(p_ref, out_ref):
    p = p_ref[...]
    m = jnp.maximum(p[0:1, :], p[1:2, :])
    out_ref[...] = (m > 0.5).astype(jnp.float32)


def _combine(partials):
    return pl.pallas_call(
        _combine_body,
        grid=(1,),
        in_specs=[pl.BlockSpec((NC, NPAD), lambda i: (0, 0))],
        out_specs=pl.BlockSpec((1, NPAD), lambda i: (0, 0)),
        out_shape=jax.ShapeDtypeStruct((1, NPAD), jnp.float32),
    )(partials)


def kernel(x, edge_index, edge_attr, W1, b1, W2, b2):
    x = x.astype(jnp.float32)
    edge_attr = edge_attr.astype(jnp.float32)
    src = edge_index[0].astype(jnp.int32)
    dst = edge_index[1].astype(jnp.int32)

    w1a = W1[:D]
    w1b = W1[D:2 * D]
    w1c = W1[2 * D:]
    w2d = (W2[:, 1] - W2[:, 0]).reshape(H, 1)
    b2d = b2[1] - b2[0]

    eye4 = jnp.eye(4, dtype=jnp.float32)
    w1c4 = jnp.kron(eye4, w1c)
    w240 = jnp.kron(eye4, W2[:, 0:1])
    w241 = jnp.kron(eye4, W2[:, 1:2])
    w2d4 = jnp.kron(eye4, w2d)
    b1t = jnp.tile(b1, 4).reshape(1, 128)
    bs = jnp.stack([b2[0], b2[1], b2d, b2d]).reshape(1, 4)

    xs, xd = _project_nodes(x, w1a, w1b)

    src4 = src.reshape(4, NW, NPCH, PCHUNK)
    dst4 = dst.reshape(4, NW, NPCH, PCHUNK)
    gs5, gd5 = _gather_edges(xs, xd, src4, dst4)
    gs_p = gs5.reshape(M, 128)
    gd_p = gd5.reshape(M, 128)
    ea4 = edge_attr.reshape(4, M, DE).transpose(1, 0, 2).reshape(M, 4 * DE)

    p0t, p1t, indt = _edge_mlp(gs_p, gd_p, ea4,
                               w1c4, b1t, w240, w241, w2d4, bs)
    edge_pred = jnp.stack([p0t.reshape(E), p1t.reshape(E)], axis=1)

    dst2 = dst.reshape(NW, EPW)
    ind2 = indt.reshape(NW, EPW)
    partials = _segment_any(dst2, ind2)

    matched01 = _combine(partials)
    matched = matched01[0, :N] != 0.0
    return edge_pred, matched


# speedup vs baseline: 1.8322x; 1.0552x over previous
import functools

import jax
import jax.numpy as jnp
from jax import lax
from jax.experimental import pallas as pl
from jax.experimental.pallas import tpu as pltpu
from jax.experimental.pallas import tpu_sc as plsc

N = 10000
E = 320000
D = 128
DE = 16
H = 32

NC = 2
NS = 16
NW = NC * NS
EPW = E // NW
CHUNK = 125
NCHUNK = EPW // CHUNK
NPAD = 10240
SLICE = NPAD // NS
M = E // 4

BN = 1000


def _proj_body(x_ref, wa_ref, wb_ref, xs_ref, xd_ref):
    xb = x_ref[...]
    xs_ref[...] = jnp.dot(xb, wa_ref[...], preferred_element_type=jnp.float32)
    xd_ref[...] = jnp.dot(xb, wb_ref[...], preferred_element_type=jnp.float32)


def _project_nodes(x, w1a, w1b):
    return pl.pallas_call(
        _proj_body,
        grid=(N // BN,),
        in_specs=[
            pl.BlockSpec((BN, D), lambda i: (i, 0)),
            pl.BlockSpec((D, H), lambda i: (0, 0)),
            pl.BlockSpec((D, H), lambda i: (0, 0)),
        ],
        out_specs=[
            pl.BlockSpec((BN, H), lambda i: (i, 0)),
            pl.BlockSpec((BN, H), lambda i: (i, 0)),
        ],
        out_shape=[
            jax.ShapeDtypeStruct((N, H), jnp.float32),
            jax.ShapeDtypeStruct((N, H), jnp.float32),
        ],
    )(x, w1a, w1b)


MPW = M // NW
PCHUNK = 125
NPCH = MPW // PCHUNK


def _gather_body(xs_hbm, xd_hbm, src4, dst4, gs_hbm, gd_hbm,
                 idxs, idxd, tmp_a, tmp_b, sem_a, sem_b, sem_wa, sem_wb):
    c = lax.axis_index("c")
    s = lax.axis_index("s")
    wid = s * NC + c
    pltpu.sync_copy(src4.at[:, wid], idxs)
    pltpu.sync_copy(dst4.at[:, wid], idxd)

    def step(j, carry):
        g_a = pltpu.async_copy(xs_hbm.at[idxs.at[0, j]], tmp_a.at[0], sem_a)
        g_b = pltpu.async_copy(xd_hbm.at[idxd.at[0, j]], tmp_b.at[0], sem_b)
        wbs = []
        for a in range(4):
            p = a % 2
            g_a.wait()
            g_b.wait()
            if a + 1 < 4:
                if wbs:
                    wa_prev, wb_prev = wbs[a - 1]
                    wa_prev.wait()
                    wb_prev.wait()
                q = (a + 1) % 2
                g_a = pltpu.async_copy(xs_hbm.at[idxs.at[a + 1, j]],
                                       tmp_a.at[q], sem_a)
                g_b = pltpu.async_copy(xd_hbm.at[idxd.at[a + 1, j]],
                                       tmp_b.at[q], sem_b)
            wbs.append((
                pltpu.async_copy(tmp_a.at[p], gs_hbm.at[wid, j, :, a], sem_wa),
                pltpu.async_copy(tmp_b.at[p], gd_hbm.at[wid, j, :, a], sem_wb),
            ))
        for wa, wb in wbs[2:]:
            wa.wait()
            wb.wait()
        return carry

    lax.fori_loop(0, NPCH, step, 0)


def _gather_edges(xs, xd, src4, dst4):
    mesh = plsc.VectorSubcoreMesh(core_axis_name="c", subcore_axis_name="s", num_cores=NC, num_subcores=NS)
    k = functools.partial(
        pl.kernel,
        mesh=mesh,
        compiler_params=pltpu.CompilerParams(use_tc_tiling_on_sc=False),
        out_type=[
            jax.ShapeDtypeStruct((NW, NPCH, PCHUNK, 4, H), jnp.float32),
            jax.ShapeDtypeStruct((NW, NPCH, PCHUNK, 4, H), jnp.float32),
        ],
        scratch_types=[
            pltpu.VMEM((4, NPCH, PCHUNK), jnp.int32),
            pltpu.VMEM((4, NPCH, PCHUNK), jnp.int32),
            pltpu.VMEM((2, PCHUNK, H), jnp.float32),
            pltpu.VMEM((2, PCHUNK, H), jnp.float32),
            pltpu.SemaphoreType.DMA,
            pltpu.SemaphoreType.DMA,
            pltpu.SemaphoreType.DMA,
            pltpu.SemaphoreType.DMA,
        ],
    )(_gather_body)
    return k(xs, xd, src4, dst4)


BP = 3200


def _mlp_body(gs_ref, gd_ref, ea_ref, w1c4_ref, b1t_ref, w240_ref,
              w241_ref, w2d4_ref, bs_ref, p0_ref, p1_ref, ind_ref):
    g = gs_ref[...] + gd_ref[...]
    pre = g + jnp.dot(ea_ref[...], w1c4_ref[...],
                      preferred_element_type=jnp.float32) + b1t_ref[...]
    h = jnp.maximum(pre, 0.0)
    tdims = (((0,), (1,)), ((), ()))
    p0_ref[...] = lax.dot_general(w240_ref[...], h, tdims,
                                  preferred_element_type=jnp.float32) + bs_ref[0, 0]
    p1_ref[...] = lax.dot_general(w241_ref[...], h, tdims,
                                  preferred_element_type=jnp.float32) + bs_ref[0, 1]
    d = lax.dot_general(w2d4_ref[...], h, tdims,
                        preferred_element_type=jnp.float32) + bs_ref[0, 2]
    score = jax.nn.sigmoid(d)
    ind_ref[...] = (score > 0.9).astype(jnp.float32)


def _edge_mlp(gs_p, gd_p, ea4, w1c4, b1t, w240, w241, w2d4, bs):
    return pl.pallas_call(
        _mlp_body,
        grid=(M // BP,),
        in_specs=[
            pl.BlockSpec((BP, 128), lambda i: (i, 0)),
            pl.BlockSpec((BP, 128), lambda i: (i, 0)),
            pl.BlockSpec((BP, 64), lambda i: (i, 0)),
            pl.BlockSpec((64, 128), lambda i: (0, 0)),
            pl.BlockSpec((1, 128), lambda i: (0, 0)),
            pl.BlockSpec((128, 4), lambda i: (0, 0)),
            pl.BlockSpec((128, 4), lambda i: (0, 0)),
            pl.BlockSpec((128, 4), lambda i: (0, 0)),
            pl.BlockSpec((1, 4), lambda i: (0, 0)),
        ],
        out_specs=[
            pl.BlockSpec((4, BP), lambda i: (0, i)),
            pl.BlockSpec((4, BP), lambda i: (0, i)),
            pl.BlockSpec((4, BP), lambda i: (0, i)),
        ],
        out_shape=[
            jax.ShapeDtypeStruct((4, M), jnp.float32),
            jax.ShapeDtypeStruct((4, M), jnp.float32),
            jax.ShapeDtypeStruct((4, M), jnp.float32),
        ],
    )(gs_p, gd_p, ea4, w1c4, b1t, w240, w241, w2d4, bs)


def _scatter_body(dst2, ind2, out_hbm, dstb, indb, hist, shared, tmprow, acc):
    c = lax.axis_index("c")
    s = lax.axis_index("s")
    wid = s * NC + c
    pltpu.sync_copy(dst2.at[wid], dstb)
    pltpu.sync_copy(ind2.at[wid], indb)

    zeros16 = jnp.zeros((16,), jnp.float32)
    ones16 = jnp.ones((16,), jnp.float32)

    def zero_step(i, carry):
        hist[pl.ds(i * 16, 16)] = zeros16
        return carry

    lax.fori_loop(0, NPAD // 16, zero_step, 0)

    def scat_step(i, carry):
        idx = dstb[pl.ds(i * 16, 16)]
        v = indb[pl.ds(i * 16, 16)]
        plsc.store_scatter(hist, [idx], ones16, mask=v > 0.5)
        return carry

    lax.fori_loop(0, EPW // 16, scat_step, 0)

    pltpu.sync_copy(hist, shared.at[s])
    plsc.subcore_barrier()

    off = s * SLICE

    def zero_acc(t, carry):
        acc[pl.ds(t * 16, 16)] = zeros16
        return carry

    lax.fori_loop(0, SLICE // 16, zero_acc, 0)

    def red_row(w, carry):
        pltpu.sync_copy(shared.at[w, pl.ds(off, SLICE)], tmprow)

        def red_col(t, carry2):
            sl = pl.ds(t * 16, 16)
            acc[sl] = jnp.maximum(acc[sl], tmprow[sl])
            return carry2

        lax.fori_loop(0, SLICE // 16, red_col, 0)
        return carry

    lax.fori_loop(0, NS, red_row, 0)
    pltpu.sync_copy(acc, out_hbm.at[c, pl.ds(off, SLICE)])


def _segment_any(dst2, ind2):
    mesh = plsc.VectorSubcoreMesh(core_axis_name="c", subcore_axis_name="s", num_cores=NC, num_subcores=NS)
    k = functools.partial(
        pl.kernel,
        mesh=mesh,
        compiler_params=pltpu.CompilerParams(
            use_tc_tiling_on_sc=False, needs_layout_passes=False),
        out_type=jax.ShapeDtypeStruct((NC, NPAD), jnp.float32),
        scratch_types=[
            pltpu.VMEM((EPW,), jnp.int32),
            pltpu.VMEM((EPW,), jnp.float32),
            pltpu.VMEM((NPAD,), jnp.float32),
            pltpu.VMEM_SHARED((NS, NPAD), jnp.float32),
            pltpu.VMEM((SLICE,), jnp.float32),
            pltpu.VMEM((SLICE,), jnp.float32),
        ],
    )(_scatter_body)
    return k(dst2, ind2)


def _combine_body(p_ref, out_ref):
    p = p_ref[...]
    m = jnp.maximum(p[0:1, :], p[1:2, :])
    out_ref[...] = (m > 0.5).astype(jnp.float32)


def _combine(partials):
    return pl.pallas_call(
        _combine_body,
        grid=(1,),
        in_specs=[pl.BlockSpec((NC, NPAD), lambda i: (0, 0))],
        out_specs=pl.BlockSpec((1, NPAD), lambda i: (0, 0)),
        out_shape=jax.ShapeDtypeStruct((1, NPAD), jnp.float32),
    )(partials)


def kernel(x, edge_index, edge_attr, W1, b1, W2, b2):
    x = x.astype(jnp.float32)
    edge_attr = edge_attr.astype(jnp.float32)
    src = edge_index[0].astype(jnp.int32)
    dst = edge_index[1].astype(jnp.int32)

    w1a = W1[:D]
    w1b = W1[D:2 * D]
    w1c = W1[2 * D:]
    w2d = (W2[:, 1] - W2[:, 0]).reshape(H, 1)
    b2d = b2[1] - b2[0]

    eye4 = jnp.eye(4, dtype=jnp.float32)
    w1c4 = jnp.kron(eye4, w1c)
    w240 = jnp.kron(eye4, W2[:, 0:1])
    w241 = jnp.kron(eye4, W2[:, 1:2])
    w2d4 = jnp.kron(eye4, w2d)
    b1t = jnp.tile(b1, 4).reshape(1, 128)
    bs = jnp.stack([b2[0], b2[1], b2d, b2d]).reshape(1, 4)

    xs, xd = _project_nodes(x, w1a, w1b)

    src4 = src.reshape(4, NW, NPCH, PCHUNK)
    dst4 = dst.reshape(4, NW, NPCH, PCHUNK)
    gs5, gd5 = _gather_edges(xs, xd, src4, dst4)
    gs_p = gs5.reshape(M, 128)
    gd_p = gd5.reshape(M, 128)
    ea4 = edge_attr.reshape(4, M, DE).transpose(1, 0, 2).reshape(M, 4 * DE)

    p0t, p1t, indt = _edge_mlp(gs_p, gd_p, ea4,
                               w1c4, b1t, w240, w241, w2d4, bs)
    edge_pred = jnp.stack([p0t.reshape(E), p1t.reshape(E)], axis=1)

    dst2 = dst.reshape(NW, EPW)
    ind2 = indt.reshape(NW, EPW)
    partials = _segment_any(dst2, ind2)

    matched01 = _combine(partials)
    matched = matched01[0, :N] != 0.0
    return edge_pred, matched


# speedup vs baseline: 1.9805x; 1.0809x over previous
import functools

import jax
import jax.numpy as jnp
from jax import lax
from jax.experimental import pallas as pl
from jax.experimental.pallas import tpu as pltpu
from jax.experimental.pallas import tpu_sc as plsc

N = 10000
E = 320000
D = 128
DE = 16
H = 32

NC = 2
NS = 16
NW = NC * NS
EPW = E // NW
CHUNK = 125
NCHUNK = EPW // CHUNK
NPAD = 10240
SLICE = NPAD // NS
M = E // 4

BN = 1000


def _proj_body(x_ref, wa_ref, wb_ref, xs_ref, xd_ref):
    xb = x_ref[...]
    xs_ref[...] = jnp.dot(xb, wa_ref[...], preferred_element_type=jnp.float32)
    xd_ref[...] = jnp.dot(xb, wb_ref[...], preferred_element_type=jnp.float32)


def _project_nodes(x, w1a, w1b):
    return pl.pallas_call(
        _proj_body,
        grid=(N // BN,),
        in_specs=[
            pl.BlockSpec((BN, D), lambda i: (i, 0)),
            pl.BlockSpec((D, H), lambda i: (0, 0)),
            pl.BlockSpec((D, H), lambda i: (0, 0)),
        ],
        out_specs=[
            pl.BlockSpec((BN, H), lambda i: (i, 0)),
            pl.BlockSpec((BN, H), lambda i: (i, 0)),
        ],
        out_shape=[
            jax.ShapeDtypeStruct((N, H), jnp.float32),
            jax.ShapeDtypeStruct((N, H), jnp.float32),
        ],
    )(x, w1a, w1b)


MPW = M // NW
PCHUNK = 125
NPCH = MPW // PCHUNK


def _gather_body(xs_hbm, xd_hbm, src4, dst4, gs_hbm, gd_hbm,
                 idxs, idxd, tmp_a, tmp_b,
                 sga0, sga1, sgb0, sgb1, swa0, swa1, swb0, swb1):
    c = lax.axis_index("c")
    s = lax.axis_index("s")
    wid = s * NC + c
    pltpu.sync_copy(src4.at[:, wid], idxs)
    pltpu.sync_copy(dst4.at[:, wid], idxd)

    sga = (sga0, sga1)
    sgb = (sgb0, sgb1)
    swa = (swa0, swa1)
    swb = (swb0, swb1)

    def step(j, carry):
        g_a = pltpu.async_copy(xs_hbm.at[idxs.at[0, j]], tmp_a.at[0], sga[0])
        g_b = pltpu.async_copy(xd_hbm.at[idxd.at[0, j]], tmp_b.at[0], sgb[0])
        wbs = []
        for a in range(4):
            p = a % 2
            g_cur = (g_a, g_b)
            if a + 1 < 4:
                if a >= 1:
                    wa_prev, wb_prev = wbs[a - 1]
                    wa_prev.wait()
                    wb_prev.wait()
                q = (a + 1) % 2
                g_a = pltpu.async_copy(xs_hbm.at[idxs.at[a + 1, j]],
                                       tmp_a.at[q], sga[q])
                g_b = pltpu.async_copy(xd_hbm.at[idxd.at[a + 1, j]],
                                       tmp_b.at[q], sgb[q])
            g_cur[0].wait()
            g_cur[1].wait()
            wbs.append((
                pltpu.async_copy(tmp_a.at[p], gs_hbm.at[wid, j, :, a], swa[p]),
                pltpu.async_copy(tmp_b.at[p], gd_hbm.at[wid, j, :, a], swb[p]),
            ))
        for wa, wb in wbs[2:]:
            wa.wait()
            wb.wait()
        return carry

    lax.fori_loop(0, NPCH, step, 0)


def _gather_edges(xs, xd, src4, dst4):
    mesh = plsc.VectorSubcoreMesh(core_axis_name="c", subcore_axis_name="s", num_cores=NC, num_subcores=NS)
    k = functools.partial(
        pl.kernel,
        mesh=mesh,
        compiler_params=pltpu.CompilerParams(use_tc_tiling_on_sc=False),
        out_type=[
            jax.ShapeDtypeStruct((NW, NPCH, PCHUNK, 4, H), jnp.float32),
            jax.ShapeDtypeStruct((NW, NPCH, PCHUNK, 4, H), jnp.float32),
        ],
        scratch_types=[
            pltpu.VMEM((4, NPCH, PCHUNK), jnp.int32),
            pltpu.VMEM((4, NPCH, PCHUNK), jnp.int32),
            pltpu.VMEM((2, PCHUNK, H), jnp.float32),
            pltpu.VMEM((2, PCHUNK, H), jnp.float32),
            pltpu.SemaphoreType.DMA,
            pltpu.SemaphoreType.DMA,
            pltpu.SemaphoreType.DMA,
            pltpu.SemaphoreType.DMA,
            pltpu.SemaphoreType.DMA,
            pltpu.SemaphoreType.DMA,
            pltpu.SemaphoreType.DMA,
            pltpu.SemaphoreType.DMA,
        ],
    )(_gather_body)
    return k(xs, xd, src4, dst4)


BP = 3200


def _mlp_body(gs_ref, gd_ref, ea_ref, w1c4_ref, b1t_ref, w240_ref,
              w241_ref, w2d4_ref, bs_ref, p0_ref, p1_ref, ind_ref):
    g = gs_ref[...] + gd_ref[...]
    pre = g + jnp.dot(ea_ref[...], w1c4_ref[...],
                      preferred_element_type=jnp.float32) + b1t_ref[...]
    h = jnp.maximum(pre, 0.0)
    tdims = (((0,), (1,)), ((), ()))
    p0_ref[...] = lax.dot_general(w240_ref[...], h, tdims,
                                  preferred_element_type=jnp.float32) + bs_ref[0, 0]
    p1_ref[...] = lax.dot_general(w241_ref[...], h, tdims,
                                  preferred_element_type=jnp.float32) + bs_ref[0, 1]
    d = lax.dot_general(w2d4_ref[...], h, tdims,
                        preferred_element_type=jnp.float32) + bs_ref[0, 2]
    score = jax.nn.sigmoid(d)
    ind_ref[...] = (score > 0.9).astype(jnp.float32)


def _edge_mlp(gs_p, gd_p, ea4, w1c4, b1t, w240, w241, w2d4, bs):
    return pl.pallas_call(
        _mlp_body,
        grid=(M // BP,),
        in_specs=[
            pl.BlockSpec((BP, 128), lambda i: (i, 0)),
            pl.BlockSpec((BP, 128), lambda i: (i, 0)),
            pl.BlockSpec((BP, 64), lambda i: (i, 0)),
            pl.BlockSpec((64, 128), lambda i: (0, 0)),
            pl.BlockSpec((1, 128), lambda i: (0, 0)),
            pl.BlockSpec((128, 4), lambda i: (0, 0)),
            pl.BlockSpec((128, 4), lambda i: (0, 0)),
            pl.BlockSpec((128, 4), lambda i: (0, 0)),
            pl.BlockSpec((1, 4), lambda i: (0, 0)),
        ],
        out_specs=[
            pl.BlockSpec((4, BP), lambda i: (0, i)),
            pl.BlockSpec((4, BP), lambda i: (0, i)),
            pl.BlockSpec((4, BP), lambda i: (0, i)),
        ],
        out_shape=[
            jax.ShapeDtypeStruct((4, M), jnp.float32),
            jax.ShapeDtypeStruct((4, M), jnp.float32),
            jax.ShapeDtypeStruct((4, M), jnp.float32),
        ],
    )(gs_p, gd_p, ea4, w1c4, b1t, w240, w241, w2d4, bs)


def _scatter_body(dst2, ind2, out_hbm, dstb, indb, hist, shared, tmprow, acc):
    c = lax.axis_index("c")
    s = lax.axis_index("s")
    wid = s * NC + c
    pltpu.sync_copy(dst2.at[wid], dstb)
    pltpu.sync_copy(ind2.at[wid], indb)

    zeros16 = jnp.zeros((16,), jnp.float32)
    ones16 = jnp.ones((16,), jnp.float32)

    def zero_step(i, carry):
        hist[pl.ds(i * 16, 16)] = zeros16
        return carry

    lax.fori_loop(0, NPAD // 16, zero_step, 0)

    def scat_step(i, carry):
        idx = dstb[pl.ds(i * 16, 16)]
        v = indb[pl.ds(i * 16, 16)]
        plsc.store_scatter(hist, [idx], ones16, mask=v > 0.5)
        return carry

    lax.fori_loop(0, EPW // 16, scat_step, 0)

    pltpu.sync_copy(hist, shared.at[s])
    plsc.subcore_barrier()

    off = s * SLICE

    def zero_acc(t, carry):
        acc[pl.ds(t * 16, 16)] = zeros16
        return carry

    lax.fori_loop(0, SLICE // 16, zero_acc, 0)

    def red_row(w, carry):
        pltpu.sync_copy(shared.at[w, pl.ds(off, SLICE)], tmprow)

        def red_col(t, carry2):
            sl = pl.ds(t * 16, 16)
            acc[sl] = jnp.maximum(acc[sl], tmprow[sl])
            return carry2

        lax.fori_loop(0, SLICE // 16, red_col, 0)
        return carry

    lax.fori_loop(0, NS, red_row, 0)
    pltpu.sync_copy(acc, out_hbm.at[c, pl.ds(off, SLICE)])


def _segment_any(dst2, ind2):
    mesh = plsc.VectorSubcoreMesh(core_axis_name="c", subcore_axis_name="s", num_cores=NC, num_subcores=NS)
    k = functools.partial(
        pl.kernel,
        mesh=mesh,
        compiler_params=pltpu.CompilerParams(
            use_tc_tiling_on_sc=False, needs_layout_passes=False),
        out_type=jax.ShapeDtypeStruct((NC, NPAD), jnp.float32),
        scratch_types=[
            pltpu.VMEM((EPW,), jnp.int32),
            pltpu.VMEM((EPW,), jnp.float32),
            pltpu.VMEM((NPAD,), jnp.float32),
            pltpu.VMEM_SHARED((NS, NPAD), jnp.float32),
            pltpu.VMEM((SLICE,), jnp.float32),
            pltpu.VMEM((SLICE,), jnp.float32),
        ],
    )(_scatter_body)
    return k(dst2, ind2)


def _combine_body(p_ref, out_ref):
    p = p_ref[...]
    m = jnp.maximum(p[0:1, :], p[1:2, :])
    out_ref[...] = (m > 0.5).astype(jnp.float32)


def _combine(partials):
    return pl.pallas_call(
        _combine_body,
        grid=(1,),
        in_specs=[pl.BlockSpec((NC, NPAD), lambda i: (0, 0))],
        out_specs=pl.BlockSpec((1, NPAD), lambda i: (0, 0)),
        out_shape=jax.ShapeDtypeStruct((1, NPAD), jnp.float32),
    )(partials)


def kernel(x, edge_index, edge_attr, W1, b1, W2, b2):
    x = x.astype(jnp.float32)
    edge_attr = edge_attr.astype(jnp.float32)
    src = edge_index[0].astype(jnp.int32)
    dst = edge_index[1].astype(jnp.int32)

    w1a = W1[:D]
    w1b = W1[D:2 * D]
    w1c = W1[2 * D:]
    w2d = (W2[:, 1] - W2[:, 0]).reshape(H, 1)
    b2d = b2[1] - b2[0]

    eye4 = jnp.eye(4, dtype=jnp.float32)
    w1c4 = jnp.kron(eye4, w1c)
    w240 = jnp.kron(eye4, W2[:, 0:1])
    w241 = jnp.kron(eye4, W2[:, 1:2])
    w2d4 = jnp.kron(eye4, w2d)
    b1t = jnp.tile(b1, 4).reshape(1, 128)
    bs = jnp.stack([b2[0], b2[1], b2d, b2d]).reshape(1, 4)

    xs, xd = _project_nodes(x, w1a, w1b)

    src4 = src.reshape(4, NW, NPCH, PCHUNK)
    dst4 = dst.reshape(4, NW, NPCH, PCHUNK)
    gs5, gd5 = _gather_edges(xs, xd, src4, dst4)
    gs_p = gs5.reshape(M, 128)
    gd_p = gd5.reshape(M, 128)
    ea4 = edge_attr.reshape(4, M, DE).transpose(1, 0, 2).reshape(M, 4 * DE)

    p0t, p1t, indt = _edge_mlp(gs_p, gd_p, ea4,
                               w1c4, b1t, w240, w241, w2d4, bs)
    edge_pred = jnp.stack([p0t.reshape(E), p1t.reshape(E)], axis=1)

    dst2 = dst.reshape(NW, EPW)
    ind2 = indt.reshape(NW, EPW)
    partials = _segment_any(dst2, ind2)

    matched01 = _combine(partials)
    matched = matched01[0, :N] != 0.0
    return edge_pred, matched
